# baseline (speedup 1.0000x reference)
import functools

import jax
import jax.numpy as jnp
from jax import lax
from jax.experimental import pallas as pl
from jax.experimental.pallas import tpu as pltpu
from jax.experimental.pallas import tpu_sc as plsc

B = 1024
C_IN = 26
C_OUT = 36
NM = 1024
NC, NS, L = 2, 16, 16
NW = NC * NS
R_PER_W = NM // NW
RSUB = 8
BSUB = 128
NBLK_R = R_PER_W // RSUB
NBLK_B = B // BSUB
NITER = NBLK_R * NBLK_B
NVEC = (RSUB * BSUB) // L
VPR = BSUB // L
NBUF = 3
GEN_OFF = 28.0
OCC_OFF = 32.0
CTAB = 200

PASS_OUT = ((0, 0, 18), (19, 18, 2), (24, 34, 2))


def _fire_in(x_hbm, in_v, k, rs, bs, sem):
    pltpu.make_async_copy(x_hbm.at[:, rs, bs], in_v.at[k], sem).start()


def _wait_in(x_hbm, in_v, k, rs, bs, sem):
    pltpu.make_async_copy(x_hbm.at[:, rs, bs], in_v.at[k], sem).wait()


def _fire_out(out_hbm, emb_v, in_v, k, rs, bs, sem):
    pltpu.make_async_copy(emb_v.at[k], out_hbm.at[pl.ds(20, 14), rs, bs], sem).start()
    for sc0, dc0, n in PASS_OUT:
        pltpu.make_async_copy(in_v.at[k, pl.ds(sc0, n)],
                              out_hbm.at[pl.ds(dc0, n), rs, bs], sem).start()


def _drain_out(out_hbm, emb_v, in_v, k, rs, bs, sem):
    pltpu.make_async_copy(emb_v.at[k], out_hbm.at[pl.ds(20, 14), rs, bs], sem).wait()
    for sc0, dc0, n in PASS_OUT:
        pltpu.make_async_copy(in_v.at[k, pl.ds(sc0, n)],
                              out_hbm.at[pl.ds(dc0, n), rs, bs], sem).wait()


def _block_slices(base_r, t):
    rs = pl.ds(base_r + (t // NBLK_B) * RSUB, RSUB)
    bs = pl.ds((t % NBLK_B) * BSUB, BSUB)
    return rs, bs


def _sc_body(x_hbm, ctab_hbm, out_hbm, ctab_v, in_v, emb_v,
             si0, si1, si2, so0, so1, so2):
    si = (si0, si1, si2)
    so = (so0, so1, so2)
    c = lax.axis_index("c")
    s = lax.axis_index("s")
    wid = s * NC + c
    base_r = wid * R_PER_W

    pltpu.sync_copy(ctab_hbm, ctab_v)

    rs0, bs0 = _block_slices(base_r, 0)
    _fire_in(x_hbm, in_v, 0, rs0, bs0, si[0])

    def compute(k):
        @plsc.parallel_loop(0, NVEC, 1, unroll=16)
        def per_vec(v):
            r = v // VPR
            sl = pl.ds((v % VPR) * L, L)
            av = in_v[k, 21, r, sl]
            gv = in_v[k, 22, r, sl]
            ov = in_v[k, 23, r, sl]
            ab = (av * 4.0).astype(jnp.int32)
            gb = (gv * 2.0 + GEN_OFF).astype(jnp.int32)
            ob = (ov * 8.0 + OCC_OFF).astype(jnp.int32)
            emb_v[k, 0, r, sl] = plsc.load_gather(ctab_v, [ab])
            for d in range(1, 4):
                emb_v[k, d, r, sl] = plsc.load_gather(ctab_v, [ab + d])
            emb_v[k, 4, r, sl] = plsc.load_gather(ctab_v, [gb])
            emb_v[k, 5, r, sl] = plsc.load_gather(ctab_v, [gb + 1])
            emb_v[k, 6, r, sl] = plsc.load_gather(ctab_v, [ob])
            for d in range(1, 8):
                emb_v[k, 6 + d, r, sl] = plsc.load_gather(ctab_v, [ob + d])

    def per_turn(g, carry):
        for k in range(NBUF):
            t = g * NBUF + k

            k2 = (k + 1) % NBUF

            @pl.when(jnp.logical_and(t >= 2, t - 2 < NITER))
            def _():
                rs, bs = _block_slices(base_r, t - 2)
                _drain_out(out_hbm, emb_v, in_v, k2, rs, bs, so[k2])

            @pl.when(t + 1 < NITER)
            def _():
                rs, bs = _block_slices(base_r, t + 1)
                _fire_in(x_hbm, in_v, k2, rs, bs, si[k2])

            @pl.when(t < NITER)
            def _():
                rs, bs = _block_slices(base_r, t)
                _wait_in(x_hbm, in_v, k, rs, bs, si[k])
                compute(k)
                _fire_out(out_hbm, emb_v, in_v, k, rs, bs, so[k])

        return carry

    lax.fori_loop(0, (NITER + NBUF) // NBUF, per_turn, 0)


@jax.jit
def kernel(x, age_table, gender_table, occupation_table):
    x_t = jnp.transpose(x.reshape(B, C_IN, NM), (1, 2, 0))
    ctab = jnp.concatenate([age_table.reshape(-1), gender_table.reshape(-1),
                            occupation_table.reshape(-1)])
    mesh = plsc.VectorSubcoreMesh(core_axis_name="c", subcore_axis_name="s",
                                  num_cores=NC, num_subcores=NS)
    out_t = pl.kernel(
        _sc_body,
        out_type=jax.ShapeDtypeStruct((C_OUT, NM, B), jnp.float32),
        mesh=mesh,
        scratch_types=[
            pltpu.VMEM((CTAB,), jnp.float32),
            pltpu.VMEM((NBUF, C_IN, RSUB, BSUB), jnp.float32),
            pltpu.VMEM((NBUF, 14, RSUB, BSUB), jnp.float32),
            pltpu.SemaphoreType.DMA,
            pltpu.SemaphoreType.DMA,
            pltpu.SemaphoreType.DMA,
            pltpu.SemaphoreType.DMA,
            pltpu.SemaphoreType.DMA,
            pltpu.SemaphoreType.DMA,
        ],
        compiler_params=pltpu.CompilerParams(use_tc_tiling_on_sc=True,
                                             needs_layout_passes=False),
    )(x_t, ctab)
    return jnp.transpose(out_t, (2, 0, 1)).reshape(B, C_OUT, 32, 32)


# speedup vs baseline: 1.0230x; 1.0230x over previous
import functools

import jax
import jax.numpy as jnp
from jax import lax
from jax.experimental import pallas as pl
from jax.experimental.pallas import tpu as pltpu
from jax.experimental.pallas import tpu_sc as plsc

B = 1024
C_IN = 26
C_OUT = 36
NM = 1024
NC, NS, L = 2, 16, 16
NW = NC * NS
R_PER_W = NM // NW
RSUB = 8
BSUB = 128
NBLK_R = R_PER_W // RSUB
NBLK_B = B // BSUB
NITER = NBLK_R * NBLK_B
NVEC = (RSUB * BSUB) // L
VPR = BSUB // L
NBUF = 3
GEN_OFF = 28.0
OCC_OFF = 32.0
CTAB = 200

PASS_OUT = ((0, 0, 18), (19, 18, 2), (24, 34, 2))


def _fire_in(x_hbm, in_v, k, rs, bs, sem):
    pltpu.make_async_copy(x_hbm.at[:, rs, bs], in_v.at[k], sem).start()


def _wait_in(x_hbm, in_v, k, rs, bs, sem):
    pltpu.make_async_copy(x_hbm.at[:, rs, bs], in_v.at[k], sem).wait()


def _fire_out(out_hbm, emb_v, in_v, k, rs, bs, sem):
    pltpu.make_async_copy(emb_v.at[k], out_hbm.at[pl.ds(20, 14), rs, bs], sem).start()
    for sc0, dc0, n in PASS_OUT:
        pltpu.make_async_copy(in_v.at[k, pl.ds(sc0, n)],
                              out_hbm.at[pl.ds(dc0, n), rs, bs], sem).start()


def _drain_out(out_hbm, emb_v, in_v, k, rs, bs, sem):
    pltpu.make_async_copy(emb_v.at[k], out_hbm.at[pl.ds(20, 14), rs, bs], sem).wait()
    for sc0, dc0, n in PASS_OUT:
        pltpu.make_async_copy(in_v.at[k, pl.ds(sc0, n)],
                              out_hbm.at[pl.ds(dc0, n), rs, bs], sem).wait()


def _block_slices(base_r, t):
    rs = pl.ds(base_r + (t // NBLK_B) * RSUB, RSUB)
    bs = pl.ds((t % NBLK_B) * BSUB, BSUB)
    return rs, bs


def _sc_body(x_hbm, ctab_hbm, out_hbm, ctab_v, in_v, emb_v,
             si0, si1, si2, so0, so1, so2):
    si = (si0, si1, si2)
    so = (so0, so1, so2)
    c = lax.axis_index("c")
    s = lax.axis_index("s")
    wid = s * NC + c
    base_r = wid * R_PER_W

    pltpu.sync_copy(ctab_hbm, ctab_v)

    rs0, bs0 = _block_slices(base_r, 0)
    _fire_in(x_hbm, in_v, 0, rs0, bs0, si[0])

    def compute(k):
        @plsc.parallel_loop(0, NVEC, 1, unroll=8)
        def per_vec(v):
            r = v // VPR
            sl = pl.ds((v % VPR) * L, L)
            av = in_v[k, 21, r, sl]
            gv = in_v[k, 22, r, sl]
            ov = in_v[k, 23, r, sl]
            ab = (av * 4.0).astype(jnp.int32)
            gb = (gv * 2.0 + GEN_OFF).astype(jnp.int32)
            ob = (ov * 8.0 + OCC_OFF).astype(jnp.int32)
            emb_v[k, 0, r, sl] = plsc.load_gather(ctab_v, [ab])
            for d in range(1, 4):
                emb_v[k, d, r, sl] = plsc.load_gather(ctab_v, [ab + d])
            emb_v[k, 4, r, sl] = plsc.load_gather(ctab_v, [gb])
            emb_v[k, 5, r, sl] = plsc.load_gather(ctab_v, [gb + 1])
            emb_v[k, 6, r, sl] = plsc.load_gather(ctab_v, [ob])
            for d in range(1, 8):
                emb_v[k, 6 + d, r, sl] = plsc.load_gather(ctab_v, [ob + d])

    def per_turn(g, carry):
        for k in range(NBUF):
            t = g * NBUF + k

            k2 = (k + 1) % NBUF

            @pl.when(jnp.logical_and(t >= 2, t - 2 < NITER))
            def _():
                rs, bs = _block_slices(base_r, t - 2)
                _drain_out(out_hbm, emb_v, in_v, k2, rs, bs, so[k2])

            @pl.when(t + 1 < NITER)
            def _():
                rs, bs = _block_slices(base_r, t + 1)
                _fire_in(x_hbm, in_v, k2, rs, bs, si[k2])

            @pl.when(t < NITER)
            def _():
                rs, bs = _block_slices(base_r, t)
                _wait_in(x_hbm, in_v, k, rs, bs, si[k])
                compute(k)
                _fire_out(out_hbm, emb_v, in_v, k, rs, bs, so[k])

        return carry

    lax.fori_loop(0, (NITER + NBUF) // NBUF, per_turn, 0)


@jax.jit
def kernel(x, age_table, gender_table, occupation_table):
    x_t = jnp.transpose(x.reshape(B, C_IN, NM), (1, 2, 0))
    ctab = jnp.concatenate([age_table.reshape(-1), gender_table.reshape(-1),
                            occupation_table.reshape(-1)])
    mesh = plsc.VectorSubcoreMesh(core_axis_name="c", subcore_axis_name="s",
                                  num_cores=NC, num_subcores=NS)
    out_t = pl.kernel(
        _sc_body,
        out_type=jax.ShapeDtypeStruct((C_OUT, NM, B), jnp.float32),
        mesh=mesh,
        scratch_types=[
            pltpu.VMEM((CTAB,), jnp.float32),
            pltpu.VMEM((NBUF, C_IN, RSUB, BSUB), jnp.float32),
            pltpu.VMEM((NBUF, 14, RSUB, BSUB), jnp.float32),
            pltpu.SemaphoreType.DMA,
            pltpu.SemaphoreType.DMA,
            pltpu.SemaphoreType.DMA,
            pltpu.SemaphoreType.DMA,
            pltpu.SemaphoreType.DMA,
            pltpu.SemaphoreType.DMA,
        ],
        compiler_params=pltpu.CompilerParams(use_tc_tiling_on_sc=True,
                                             needs_layout_passes=False),
    )(x_t, ctab)
    return jnp.transpose(out_t, (2, 0, 1)).reshape(B, C_OUT, 32, 32)
